# Initial kernel scaffold; baseline (speedup 1.0000x reference)
import jax
import jax.numpy as jnp
from jax.experimental import pallas as pl


def kernel(atom_type, residue_type, W_atom, W_res):
    raise NotImplementedError("write your pallas kernel here")


# speedup vs baseline: 1.0263x; 1.0263x over previous
import functools

import jax
import jax.numpy as jnp
from jax import lax
from jax.experimental import pallas as pl
from jax.experimental.pallas import tpu as pltpu
from jax.experimental.pallas import tpu_sc as plsc

_NC = 2
_NS = 16
_NW = _NC * _NS
_CHUNK = 128


def _build(B_pad, n_chunks, D, dtype):
    mesh = plsc.VectorSubcoreMesh(core_axis_name="c", subcore_axis_name="s")
    out_type = jax.ShapeDtypeStruct((2 * B_pad, D), dtype)

    @functools.partial(
        pl.kernel,
        mesh=mesh,
        out_type=out_type,
        compiler_params=pltpu.CompilerParams(use_tc_tiling_on_sc=False),
        scratch_types=[
            pltpu.VMEM((n_chunks, _CHUNK), jnp.int32),
            pltpu.VMEM((n_chunks, _CHUNK), jnp.int32),
            pltpu.VMEM((n_chunks, _CHUNK), jnp.int32),
            pltpu.VMEM((n_chunks, _CHUNK), jnp.int32),
            pltpu.VMEM((_CHUNK, D), dtype),
            pltpu.VMEM((_CHUNK, D), dtype),
            pltpu.SemaphoreType.DMA,
            pltpu.SemaphoreType.DMA,
        ],
    )
    def k(idx_a_hbm, idx_r_hbm, oidx_a_hbm, oidx_r_hbm, w_atom_hbm, w_res_hbm,
          out_hbm, idx_a_v, idx_r_v, oidx_a_v, oidx_r_v, rows_a, rows_r,
          sem_a, sem_r):
        wid = lax.axis_index("s") * _NC + lax.axis_index("c")
        pltpu.sync_copy(idx_a_hbm.at[wid], idx_a_v)
        pltpu.sync_copy(idx_r_hbm.at[wid], idx_r_v)
        pltpu.sync_copy(oidx_a_hbm.at[wid], oidx_a_v)
        pltpu.sync_copy(oidx_r_hbm.at[wid], oidx_r_v)

        def body(j, carry):
            cp_a = pltpu.async_copy(w_atom_hbm.at[idx_a_v.at[j]], rows_a, sem_a)
            cp_r = pltpu.async_copy(w_res_hbm.at[idx_r_v.at[j]], rows_r, sem_r)
            cp_a.wait()
            cp_r.wait()
            st_a = pltpu.async_copy(rows_a, out_hbm.at[oidx_a_v.at[j]], sem_a)
            st_r = pltpu.async_copy(rows_r, out_hbm.at[oidx_r_v.at[j]], sem_r)
            st_a.wait()
            st_r.wait()
            return carry

        lax.fori_loop(0, n_chunks, body, 0)

    return k


def kernel(atom_type, residue_type, W_atom, W_res):
    B = atom_type.shape[0]
    D = W_atom.shape[1]
    per_w = _NW * _CHUNK
    B_pad = ((B + per_w - 1) // per_w) * per_w
    n_chunks = B_pad // per_w

    idx_a = jnp.zeros((B_pad,), jnp.int32).at[:B].set(atom_type.astype(jnp.int32))
    idx_r = jnp.zeros((B_pad,), jnp.int32).at[:B].set(residue_type.astype(jnp.int32))
    idx_a = idx_a.reshape(_NW, n_chunks, _CHUNK)
    idx_r = idx_r.reshape(_NW, n_chunks, _CHUNK)
    rows = jnp.arange(B_pad, dtype=jnp.int32).reshape(_NW, n_chunks, _CHUNK)
    oidx_a = rows * 2
    oidx_r = rows * 2 + 1

    out = _build(B_pad, n_chunks, D, W_atom.dtype)(
        idx_a, idx_r, oidx_a, oidx_r, W_atom, W_res)
    return out.reshape(B_pad, 2 * D)[:B]


# speedup vs baseline: 1.0444x; 1.0177x over previous
import functools

import jax
import jax.numpy as jnp
from jax import lax
from jax.experimental import pallas as pl
from jax.experimental.pallas import tpu as pltpu
from jax.experimental.pallas import tpu_sc as plsc

_NC = 2
_NS = 16
_NW = _NC * _NS
_CHUNK = 128
_NB = 5
_P = 2


def _build(B_pad, n_chunks, D, dtype):
    mesh = plsc.VectorSubcoreMesh(core_axis_name="c", subcore_axis_name="s")
    out_type = jax.ShapeDtypeStruct((2 * B_pad, D), dtype)

    @functools.partial(
        pl.kernel,
        mesh=mesh,
        out_type=out_type,
        compiler_params=pltpu.CompilerParams(use_tc_tiling_on_sc=False),
        scratch_types=[
            pltpu.VMEM((n_chunks, _CHUNK), jnp.int32),
            pltpu.VMEM((n_chunks, _CHUNK), jnp.int32),
            pltpu.VMEM((n_chunks, _CHUNK), jnp.int32),
            pltpu.VMEM((n_chunks, _CHUNK), jnp.int32),
            pltpu.VMEM((_NB, _CHUNK, D), dtype),
            pltpu.VMEM((_NB, _CHUNK, D), dtype),
            pltpu.SemaphoreType.DMA,
            pltpu.SemaphoreType.DMA((_NB,)),
            pltpu.SemaphoreType.DMA((_NB,)),
        ],
    )
    def k(idx_a_hbm, idx_r_hbm, oidx_a_hbm, oidx_r_hbm, w_atom_hbm, w_res_hbm,
          out_hbm, idx_a_v, idx_r_v, oidx_a_v, oidx_r_v, rows_a, rows_r,
          sem_i, sem_g, sem_s):
        wid = lax.axis_index("s") * _NC + lax.axis_index("c")
        pltpu.async_copy(idx_a_hbm.at[wid], idx_a_v, sem_i)
        pltpu.async_copy(idx_r_hbm.at[wid], idx_r_v, sem_i)
        pltpu.async_copy(oidx_a_hbm.at[wid], oidx_a_v, sem_i)
        cp_i = pltpu.async_copy(oidx_r_hbm.at[wid], oidx_r_v, sem_i)
        pltpu.make_async_copy(idx_a_hbm.at[wid], idx_a_v, sem_i).wait()
        pltpu.make_async_copy(idx_r_hbm.at[wid], idx_r_v, sem_i).wait()
        pltpu.make_async_copy(oidx_a_hbm.at[wid], oidx_a_v, sem_i).wait()
        cp_i.wait()

        def gather(j, b):
            pltpu.async_copy(w_atom_hbm.at[idx_a_v.at[j]], rows_a.at[b],
                             sem_g.at[b])
            pltpu.async_copy(w_res_hbm.at[idx_r_v.at[j]], rows_r.at[b],
                             sem_g.at[b])

        def wait_gather(b):
            pltpu.make_async_copy(w_atom_hbm.at[idx_a_v.at[0]], rows_a.at[b],
                                  sem_g.at[b]).wait()
            pltpu.make_async_copy(w_res_hbm.at[idx_r_v.at[0]], rows_r.at[b],
                                  sem_g.at[b]).wait()

        def scatter(j, b):
            pltpu.async_copy(rows_a.at[b], out_hbm.at[oidx_a_v.at[j]],
                             sem_s.at[b])
            pltpu.async_copy(rows_r.at[b], out_hbm.at[oidx_r_v.at[j]],
                             sem_s.at[b])

        def wait_scatter(b):
            pltpu.make_async_copy(rows_a.at[b], out_hbm.at[oidx_a_v.at[0]],
                                  sem_s.at[b]).wait()
            pltpu.make_async_copy(rows_r.at[b], out_hbm.at[oidx_r_v.at[0]],
                                  sem_s.at[b]).wait()

        for b in range(_P):
            gather(b, b)

        n_outer = n_chunks // _NB

        def outer(jo, carry):
            j0 = jo * _NB
            for b in range(_NB):
                j = j0 + b
                wait_gather(b)
                scatter(j, b)
                bp = (b + _P) % _NB
                jn = j + _P

                @pl.when(jn >= _NB)
                def _():
                    wait_scatter(bp)

                @pl.when(jn < n_chunks)
                def _():
                    gather(jn, bp)

            return carry

        lax.fori_loop(0, n_outer, outer, 0)

        for j in range(n_chunks - (_NB - _P), n_chunks):
            wait_scatter(j % _NB)

    return k


def kernel(atom_type, residue_type, W_atom, W_res):
    B = atom_type.shape[0]
    D = W_atom.shape[1]
    per_w = _NW * _CHUNK
    B_pad = ((B + per_w - 1) // per_w) * per_w
    n_chunks = B_pad // per_w

    idx_a = jnp.zeros((B_pad,), jnp.int32).at[:B].set(atom_type.astype(jnp.int32))
    idx_r = jnp.zeros((B_pad,), jnp.int32).at[:B].set(residue_type.astype(jnp.int32))
    idx_a = idx_a.reshape(_NW, n_chunks, _CHUNK)
    idx_r = idx_r.reshape(_NW, n_chunks, _CHUNK)
    rows = jnp.arange(B_pad, dtype=jnp.int32).reshape(_NW, n_chunks, _CHUNK)
    oidx_a = rows * 2
    oidx_r = rows * 2 + 1

    out = _build(B_pad, n_chunks, D, W_atom.dtype)(
        idx_a, idx_r, oidx_a, oidx_r, W_atom, W_res)
    return out.reshape(B_pad, 2 * D)[:B]


# speedup vs baseline: 3.6617x; 3.5060x over previous
import functools

import jax
import jax.numpy as jnp
from jax import lax
from jax.experimental import pallas as pl
from jax.experimental.pallas import tpu as pltpu
from jax.experimental.pallas import tpu_sc as plsc

_NC = 2
_NS = 16
_NW = _NC * _NS
_CHUNK = 128
_NB = 5
_P = 2


def _build(B, per_w, n_chunks, D, dtype):
    slab = n_chunks * _CHUNK
    mesh = plsc.VectorSubcoreMesh(core_axis_name="c", subcore_axis_name="s")
    out_type = jax.ShapeDtypeStruct((2 * B, D), dtype)

    @functools.partial(
        pl.kernel,
        mesh=mesh,
        out_type=out_type,
        compiler_params=pltpu.CompilerParams(use_tc_tiling_on_sc=False),
        scratch_types=[
            pltpu.VMEM((slab,), jnp.int32),
            pltpu.VMEM((slab,), jnp.int32),
            pltpu.VMEM((n_chunks, _CHUNK), jnp.int32),
            pltpu.VMEM((n_chunks, _CHUNK), jnp.int32),
            pltpu.VMEM((_NB, _CHUNK, D), dtype),
            pltpu.VMEM((_NB, _CHUNK, D), dtype),
            pltpu.SemaphoreType.DMA,
            pltpu.SemaphoreType.DMA((_NB,)),
            pltpu.SemaphoreType.DMA((_NB,)),
        ],
    )
    def k(idx_a_hbm, idx_r_hbm, oidx_a_hbm, oidx_r_hbm, w_atom_hbm, w_res_hbm,
          out_hbm, idx_a_v, idx_r_v, oidx_a_v, oidx_r_v, rows_a, rows_r,
          sem_i, sem_g, sem_s):
        wid = lax.axis_index("s") * _NC + lax.axis_index("c")
        base = jnp.minimum(wid * per_w, B - slab)

        pltpu.async_copy(idx_a_hbm.at[pl.ds(base, slab)], idx_a_v, sem_i)
        pltpu.async_copy(idx_r_hbm.at[pl.ds(base, slab)], idx_r_v, sem_i)
        pltpu.async_copy(oidx_a_hbm.at[wid], oidx_a_v, sem_i)
        pltpu.async_copy(oidx_r_hbm.at[wid], oidx_r_v, sem_i)
        pltpu.make_async_copy(idx_a_hbm.at[pl.ds(0, slab)], idx_a_v, sem_i).wait()
        pltpu.make_async_copy(idx_r_hbm.at[pl.ds(0, slab)], idx_r_v, sem_i).wait()
        pltpu.make_async_copy(oidx_a_hbm.at[0], oidx_a_v, sem_i).wait()
        pltpu.make_async_copy(oidx_r_hbm.at[0], oidx_r_v, sem_i).wait()

        def gather(j, b):
            pltpu.async_copy(w_atom_hbm.at[idx_a_v.at[pl.ds(j * _CHUNK, _CHUNK)]],
                             rows_a.at[b], sem_g.at[b])
            pltpu.async_copy(w_res_hbm.at[idx_r_v.at[pl.ds(j * _CHUNK, _CHUNK)]],
                             rows_r.at[b], sem_g.at[b])

        def wait_gather(b):
            pltpu.make_async_copy(w_atom_hbm.at[idx_a_v.at[pl.ds(0, _CHUNK)]],
                                  rows_a.at[b], sem_g.at[b]).wait()
            pltpu.make_async_copy(w_res_hbm.at[idx_r_v.at[pl.ds(0, _CHUNK)]],
                                  rows_r.at[b], sem_g.at[b]).wait()

        def scatter(j, b):
            pltpu.async_copy(rows_a.at[b], out_hbm.at[oidx_a_v.at[j]],
                             sem_s.at[b])
            pltpu.async_copy(rows_r.at[b], out_hbm.at[oidx_r_v.at[j]],
                             sem_s.at[b])

        def wait_scatter(b):
            pltpu.make_async_copy(rows_a.at[b], out_hbm.at[oidx_a_v.at[0]],
                                  sem_s.at[b]).wait()
            pltpu.make_async_copy(rows_r.at[b], out_hbm.at[oidx_r_v.at[0]],
                                  sem_s.at[b]).wait()

        for b in range(_P):
            gather(b, b)

        n_outer = n_chunks // _NB

        def outer(jo, carry):
            j0 = jo * _NB
            for b in range(_NB):
                j = j0 + b
                wait_gather(b)
                scatter(j, b)
                bp = (b + _P) % _NB
                jn = j + _P

                @pl.when(jn >= _NB)
                def _():
                    wait_scatter(bp)

                @pl.when(jn < n_chunks)
                def _():
                    gather(jn, bp)

            return carry

        lax.fori_loop(0, n_outer, outer, 0)

        for j in range(n_chunks - (_NB - _P), n_chunks):
            wait_scatter(j % _NB)

    return k


def kernel(atom_type, residue_type, W_atom, W_res):
    B = atom_type.shape[0]
    D = W_atom.shape[1]
    per_w = -(-B // _NW)
    per_w = -(-per_w // 8) * 8
    n_chunks = -(-per_w // _CHUNK)
    n_chunks = -(-n_chunks // _NB) * _NB
    slab = n_chunks * _CHUNK

    bases = jnp.minimum(jnp.arange(_NW, dtype=jnp.int32) * per_w, B - slab)
    pos = bases[:, None] + jnp.arange(slab, dtype=jnp.int32)[None, :]
    pos = pos.reshape(_NW, n_chunks, _CHUNK)
    oidx_a = pos * 2
    oidx_r = pos * 2 + 1

    out = _build(B, per_w, n_chunks, D, W_atom.dtype)(
        atom_type.astype(jnp.int32), residue_type.astype(jnp.int32),
        oidx_a, oidx_r, W_atom, W_res)
    return out.reshape(B, 2 * D)
